# Initial kernel scaffold; baseline (speedup 1.0000x reference)
import jax
import jax.numpy as jnp
from jax import lax
from jax.experimental import pallas as pl
from jax.experimental.pallas import tpu as pltpu
from jax.experimental.pallas import tpu_sc as plsc

N = 10000
E = 320000
D_IN = 128
DH = 64
NP = 10240
RPT = NP // 16
CH = 128
KA = -(-E // (16 * CH))
EA = 16 * KA * CH
KD = -(-E // (32 * CH))
ED = 32 * KD * CH


def _sc_deg_body(dstd, ones_hbm, zeros_hbm, degacc_out, idx_v, ones_v, zrow_v, acc):
    c = lax.axis_index("c")
    s = lax.axis_index("s")
    w = s * 2 + c
    pltpu.sync_copy(dstd.at[w], idx_v)
    pltpu.sync_copy(ones_hbm, ones_v)
    pltpu.sync_copy(zeros_hbm, zrow_v)
    pltpu.sync_copy(zrow_v, acc.at[pl.ds(s * RPT, RPT)])
    plsc.subcore_barrier()

    def chunk(j, carry):
        pltpu.sync_copy(ones_v, acc.at[idx_v.at[j]], add=True)
        return carry

    lax.fori_loop(0, KD, chunk, 0)
    plsc.subcore_barrier()
    pltpu.sync_copy(acc.at[pl.ds(s * RPT, RPT)],
                    degacc_out.at[c, pl.ds(s * RPT, RPT)])


def _sc_agg_body(table, srcd, dstd, zeros_hbm, out, src_v, dst_v, gbuf, zbuf, acc):
    c = lax.axis_index("c")
    s = lax.axis_index("s")
    pltpu.sync_copy(srcd.at[c, s], src_v)
    pltpu.sync_copy(dstd.at[s], dst_v)
    pltpu.sync_copy(zeros_hbm, zbuf)
    pltpu.sync_copy(zbuf, acc.at[pl.ds(s * RPT, RPT)])
    plsc.subcore_barrier()

    def chunk(j, carry):
        pltpu.sync_copy(table.at[src_v.at[j]], gbuf)
        pltpu.sync_copy(gbuf, acc.at[dst_v.at[j]], add=True)
        return carry

    lax.fori_loop(0, KA, chunk, 0)
    plsc.subcore_barrier()
    pltpu.sync_copy(acc.at[pl.ds(s * RPT, RPT)],
                    out.at[c, pl.ds(s * RPT, RPT)])


def _make_sc_deg():
    return pl.kernel(
        _sc_deg_body,
        out_type=jax.ShapeDtypeStruct((2, NP, 16), jnp.float32),
        mesh=plsc.VectorSubcoreMesh(core_axis_name="c", subcore_axis_name="s"),
        scratch_types=[
            pltpu.VMEM((KD, CH), jnp.int32),
            pltpu.VMEM((CH, 16), jnp.float32),
            pltpu.VMEM((RPT, 16), jnp.float32),
            pltpu.VMEM_SHARED((NP, 16), jnp.float32),
        ],
    )


def _make_sc_agg():
    return pl.kernel(
        _sc_agg_body,
        out_type=jax.ShapeDtypeStruct((2, NP, DH), jnp.float32),
        mesh=plsc.VectorSubcoreMesh(core_axis_name="c", subcore_axis_name="s"),
        scratch_types=[
            pltpu.VMEM((KA, CH), jnp.int32),
            pltpu.VMEM((KA, CH), jnp.int32),
            pltpu.VMEM((CH, DH), jnp.float32),
            pltpu.VMEM((RPT, DH), jnp.float32),
            pltpu.VMEM_SHARED((NP, DH), jnp.float32),
        ],
    )


def _dis(da_ref):
    deg = da_ref[0][:, 0:1] + da_ref[1][:, 0:1] + 1.0
    return lax.rsqrt(deg)


def _tc_scale_body(da_ref, xp_ref, xs_ref):
    dis = _dis(da_ref)
    xsb = xp_ref[...] * dis
    xs_ref[0] = xsb[:, :DH]
    xs_ref[1] = xsb[:, DH:]


def _tc_hidden_body(agg_ref, xs_ref, da_ref, w1_ref, b1_ref, hs_ref):
    dis = _dis(da_ref)
    ax = jnp.concatenate([agg_ref[0] + xs_ref[0], agg_ref[1] + xs_ref[1]],
                         axis=1) * dis
    h = jnp.maximum(
        jnp.dot(ax, w1_ref[...], preferred_element_type=jnp.float32)
        + b1_ref[...], 0.0)
    hs = h * dis
    hs_ref[0] = hs[:, :DH]
    hs_ref[1] = hs[:, DH:]


def _tc_out_body(agg_ref, hs_ref, da_ref, wmu_ref, bmu_ref, wls_ref, bls_ref,
                 mu_ref, ls_ref):
    dis = _dis(da_ref)
    ah = jnp.concatenate([agg_ref[0] + hs_ref[0], agg_ref[1] + hs_ref[1]],
                         axis=1) * dis
    mu_ref[...] = jnp.dot(ah, wmu_ref[...],
                          preferred_element_type=jnp.float32) + bmu_ref[...]
    ls_ref[...] = jnp.dot(ah, wls_ref[...],
                          preferred_element_type=jnp.float32) + bls_ref[...]


_B = 2048
_G = NP // _B


def _spec3(b, d):
    return pl.BlockSpec((2, b, d), lambda i: (0, i, 0))


def _make_tc_scale():
    return pl.pallas_call(
        _tc_scale_body,
        grid=(_G,),
        in_specs=[_spec3(_B, 16), pl.BlockSpec((_B, D_IN), lambda i: (i, 0))],
        out_specs=_spec3(_B, DH),
        out_shape=jax.ShapeDtypeStruct((2, NP, DH), jnp.float32),
    )


def _make_tc_hidden():
    return pl.pallas_call(
        _tc_hidden_body,
        grid=(_G,),
        in_specs=[_spec3(_B, DH), _spec3(_B, DH), _spec3(_B, 16),
                  pl.BlockSpec((D_IN, D_IN), lambda i: (0, 0)),
                  pl.BlockSpec((1, D_IN), lambda i: (0, 0))],
        out_specs=_spec3(_B, DH),
        out_shape=jax.ShapeDtypeStruct((2, NP, DH), jnp.float32),
    )


def _make_tc_out():
    return pl.pallas_call(
        _tc_out_body,
        grid=(_G,),
        in_specs=[_spec3(_B, DH), _spec3(_B, DH), _spec3(_B, 16),
                  pl.BlockSpec((D_IN, DH), lambda i: (0, 0)),
                  pl.BlockSpec((1, DH), lambda i: (0, 0)),
                  pl.BlockSpec((D_IN, DH), lambda i: (0, 0)),
                  pl.BlockSpec((1, DH), lambda i: (0, 0))],
        out_specs=[pl.BlockSpec((_B, DH), lambda i: (i, 0)),
                   pl.BlockSpec((_B, DH), lambda i: (i, 0))],
        out_shape=[jax.ShapeDtypeStruct((NP, DH), jnp.float32),
                   jax.ShapeDtypeStruct((NP, DH), jnp.float32)],
    )


def kernel(x, edge_index, W1, b1, Wmu, bmu, Wls, bls):
    f32 = jnp.float32
    src = edge_index[0]
    dst = edge_index[1]
    xp = jnp.zeros((NP, D_IN), f32).at[:N].set(x)
    pad_a = jnp.full((EA - E,), N, jnp.int32)
    srcd = jnp.concatenate([src, pad_a]).reshape(16, KA, CH)
    srcd2 = jnp.stack([srcd, srcd + NP])
    dstd = jnp.concatenate([dst, pad_a]).reshape(16, KA, CH)
    pad_d = jnp.full((ED - E,), N, jnp.int32)
    dstd32 = jnp.concatenate([dst, pad_d]).reshape(32, KD, CH)
    ones16 = jnp.ones((CH, 16), f32)
    zeros16 = jnp.zeros((RPT, 16), f32)
    zeros64 = jnp.zeros((RPT, DH), f32)

    sc_deg = _make_sc_deg()
    sc_agg = _make_sc_agg()
    tc_scale = _make_tc_scale()
    tc_hidden = _make_tc_hidden()
    tc_out = _make_tc_out()

    degacc = sc_deg(dstd32, ones16, zeros16)
    xs = tc_scale(degacc, xp)
    agg1 = sc_agg(xs.reshape(2 * NP, DH), srcd2, dstd, zeros64)
    hs = tc_hidden(agg1, xs, degacc, W1, b1.reshape(1, D_IN))
    agg2 = sc_agg(hs.reshape(2 * NP, DH), srcd2, dstd, zeros64)
    mu, ls = tc_out(agg2, hs, degacc, Wmu, bmu.reshape(1, DH),
                    Wls, bls.reshape(1, DH))
    mu = mu[:N]
    ls = ls[:N]
    return (mu, mu, ls)


# speedup vs baseline: 18.1020x; 18.1020x over previous
import jax
import jax.numpy as jnp
from jax import lax
from jax.experimental import pallas as pl
from jax.experimental.pallas import tpu as pltpu
from jax.experimental.pallas import tpu_sc as plsc

N = 10000
E = 320000
D_IN = 128
DH = 64
NP = 10240
RPT = NP // 16
CH = 128
KA = -(-E // (16 * CH))
EA = 16 * KA * CH
KD = -(-E // (32 * CH))
ED = 32 * KD * CH


def _sc_deg_body(dstd, ones_hbm, zeros_hbm, degacc_out, idx_v, ones_v, zrow_v, acc):
    c = lax.axis_index("c")
    s = lax.axis_index("s")
    w = s * 2 + c
    pltpu.sync_copy(dstd.at[w], idx_v)
    pltpu.sync_copy(ones_hbm, ones_v)
    pltpu.sync_copy(zeros_hbm, zrow_v)
    pltpu.sync_copy(zrow_v, acc.at[pl.ds(s * RPT, RPT)])
    plsc.subcore_barrier()

    def chunk(j, carry):
        pltpu.sync_copy(ones_v, acc.at[idx_v.at[j]], add=True)
        return carry

    lax.fori_loop(0, KD, chunk, 0)
    plsc.subcore_barrier()
    pltpu.sync_copy(acc.at[pl.ds(s * RPT, RPT)],
                    degacc_out.at[c, pl.ds(s * RPT, RPT)])


def _sc_agg_body(table, srcd, dstd, zeros_hbm, out, src_v, dst_v, gbuf, zbuf, acc):
    c = lax.axis_index("c")
    s = lax.axis_index("s")
    pltpu.sync_copy(srcd.at[c, s], src_v)
    pltpu.sync_copy(dstd.at[s], dst_v)
    pltpu.sync_copy(zeros_hbm, zbuf)
    pltpu.sync_copy(zbuf, acc.at[pl.ds(s * RPT, RPT)])
    plsc.subcore_barrier()

    def chunk(j, carry):
        pltpu.sync_copy(table.at[src_v.at[j]], gbuf)
        pltpu.sync_copy(gbuf, acc.at[dst_v.at[j]], add=True)
        return carry

    lax.fori_loop(0, KA, chunk, 0)
    plsc.subcore_barrier()
    pltpu.sync_copy(acc.at[pl.ds(s * RPT, RPT)],
                    out.at[c, pl.ds(s * RPT, RPT)])


def _make_sc_deg():
    return pl.kernel(
        _sc_deg_body,
        out_type=jax.ShapeDtypeStruct((2, NP, 16), jnp.float32),
        mesh=plsc.VectorSubcoreMesh(core_axis_name="c", subcore_axis_name="s"),
        compiler_params=pltpu.CompilerParams(use_tc_tiling_on_sc=False),
        scratch_types=[
            pltpu.VMEM((KD, CH), jnp.int32),
            pltpu.VMEM((CH, 16), jnp.float32),
            pltpu.VMEM((RPT, 16), jnp.float32),
            pltpu.VMEM_SHARED((NP, 16), jnp.float32),
        ],
    )


def _make_sc_agg():
    return pl.kernel(
        _sc_agg_body,
        out_type=jax.ShapeDtypeStruct((2, NP, DH), jnp.float32),
        mesh=plsc.VectorSubcoreMesh(core_axis_name="c", subcore_axis_name="s"),
        compiler_params=pltpu.CompilerParams(use_tc_tiling_on_sc=False),
        scratch_types=[
            pltpu.VMEM((KA, CH), jnp.int32),
            pltpu.VMEM((KA, CH), jnp.int32),
            pltpu.VMEM((CH, DH), jnp.float32),
            pltpu.VMEM((RPT, DH), jnp.float32),
            pltpu.VMEM_SHARED((NP, DH), jnp.float32),
        ],
    )


def _dis(da_ref):
    deg = da_ref[0][:, 0:1] + da_ref[1][:, 0:1] + 1.0
    return lax.rsqrt(deg)


def _tc_scale_body(da_ref, xp_ref, xs_ref):
    dis = _dis(da_ref)
    xsb = xp_ref[...] * dis
    xs_ref[0] = xsb[:, :DH]
    xs_ref[1] = xsb[:, DH:]


def _tc_hidden_body(agg_ref, xs_ref, da_ref, w1_ref, b1_ref, hs_ref):
    dis = _dis(da_ref)
    ax = jnp.concatenate([agg_ref[0] + xs_ref[0], agg_ref[1] + xs_ref[1]],
                         axis=1) * dis
    h = jnp.maximum(
        jnp.dot(ax, w1_ref[...], preferred_element_type=jnp.float32)
        + b1_ref[...], 0.0)
    hs = h * dis
    hs_ref[0] = hs[:, :DH]
    hs_ref[1] = hs[:, DH:]


def _tc_out_body(agg_ref, hs_ref, da_ref, wmu_ref, bmu_ref, wls_ref, bls_ref,
                 mu_ref, ls_ref):
    dis = _dis(da_ref)
    ah = jnp.concatenate([agg_ref[0] + hs_ref[0], agg_ref[1] + hs_ref[1]],
                         axis=1) * dis
    mu_ref[...] = jnp.dot(ah, wmu_ref[...],
                          preferred_element_type=jnp.float32) + bmu_ref[...]
    ls_ref[...] = jnp.dot(ah, wls_ref[...],
                          preferred_element_type=jnp.float32) + bls_ref[...]


_B = 2048
_G = NP // _B


def _spec3(b, d):
    return pl.BlockSpec((2, b, d), lambda i: (0, i, 0))


def _make_tc_scale():
    return pl.pallas_call(
        _tc_scale_body,
        grid=(_G,),
        in_specs=[_spec3(_B, 16), pl.BlockSpec((_B, D_IN), lambda i: (i, 0))],
        out_specs=_spec3(_B, DH),
        out_shape=jax.ShapeDtypeStruct((2, NP, DH), jnp.float32),
    )


def _make_tc_hidden():
    return pl.pallas_call(
        _tc_hidden_body,
        grid=(_G,),
        in_specs=[_spec3(_B, DH), _spec3(_B, DH), _spec3(_B, 16),
                  pl.BlockSpec((D_IN, D_IN), lambda i: (0, 0)),
                  pl.BlockSpec((1, D_IN), lambda i: (0, 0))],
        out_specs=_spec3(_B, DH),
        out_shape=jax.ShapeDtypeStruct((2, NP, DH), jnp.float32),
    )


def _make_tc_out():
    return pl.pallas_call(
        _tc_out_body,
        grid=(_G,),
        in_specs=[_spec3(_B, DH), _spec3(_B, DH), _spec3(_B, 16),
                  pl.BlockSpec((D_IN, DH), lambda i: (0, 0)),
                  pl.BlockSpec((1, DH), lambda i: (0, 0)),
                  pl.BlockSpec((D_IN, DH), lambda i: (0, 0)),
                  pl.BlockSpec((1, DH), lambda i: (0, 0))],
        out_specs=[pl.BlockSpec((_B, DH), lambda i: (i, 0)),
                   pl.BlockSpec((_B, DH), lambda i: (i, 0))],
        out_shape=[jax.ShapeDtypeStruct((NP, DH), jnp.float32),
                   jax.ShapeDtypeStruct((NP, DH), jnp.float32)],
    )


def kernel(x, edge_index, W1, b1, Wmu, bmu, Wls, bls):
    f32 = jnp.float32
    src = edge_index[0]
    dst = edge_index[1]
    xp = jnp.zeros((NP, D_IN), f32).at[:N].set(x)
    pad_a = jnp.full((EA - E,), N, jnp.int32)
    srcd = jnp.concatenate([src, pad_a]).reshape(16, KA, CH)
    srcd2 = jnp.stack([srcd, srcd + NP])
    dstd = jnp.concatenate([dst, pad_a]).reshape(16, KA, CH)
    pad_d = jnp.full((ED - E,), N, jnp.int32)
    dstd32 = jnp.concatenate([dst, pad_d]).reshape(32, KD, CH)
    ones16 = jnp.ones((CH, 16), f32)
    zeros16 = jnp.zeros((RPT, 16), f32)
    zeros64 = jnp.zeros((RPT, DH), f32)

    sc_deg = _make_sc_deg()
    sc_agg = _make_sc_agg()
    tc_scale = _make_tc_scale()
    tc_hidden = _make_tc_hidden()
    tc_out = _make_tc_out()

    degacc = sc_deg(dstd32, ones16, zeros16)
    xs = tc_scale(degacc, xp)
    agg1 = sc_agg(xs.reshape(2 * NP, DH), srcd2, dstd, zeros64)
    hs = tc_hidden(agg1, xs, degacc, W1, b1.reshape(1, D_IN))
    agg2 = sc_agg(hs.reshape(2 * NP, DH), srcd2, dstd, zeros64)
    mu, ls = tc_out(agg2, hs, degacc, Wmu, bmu.reshape(1, DH),
                    Wls, bls.reshape(1, DH))
    mu = mu[:N]
    ls = ls[:N]
    return (mu, mu, ls)
